# baseline (speedup 1.0000x reference)
import functools

import jax
import jax.numpy as jnp
from jax import lax
from jax.experimental import pallas as pl
from jax.experimental.pallas import tpu as pltpu
from jax.experimental.pallas import tpu_sc as plsc

_LANES = 16
_CHUNK = 128


@functools.lru_cache(maxsize=None)
def _build(B, V, T):
    info = plsc.get_sparse_core_info()
    NC, NS = info.num_cores, info.num_subcores
    NW = NC * NS
    RPW = B // NW
    NCH = T // _CHUNK
    G = 16
    NG = RPW // G
    VPC = _CHUNK // _LANES

    mesh = plsc.VectorSubcoreMesh(core_axis_name="c", subcore_axis_name="s")

    @functools.partial(
        pl.kernel,
        out_type=jax.ShapeDtypeStruct((B,), jnp.int32),
        mesh=mesh,
        compiler_params=pltpu.CompilerParams(
            needs_layout_passes=False, use_tc_tiling_on_sc=False),
        scratch_types=[
            pltpu.VMEM((NCH, _CHUNK), jnp.int32),
            pltpu.VMEM((T,), jnp.float32),
            pltpu.VMEM((_LANES,), jnp.float32),
            pltpu.VMEM((G * NCH, _CHUNK), jnp.float32),
            pltpu.VMEM((G * _LANES,), jnp.float32),
            pltpu.VMEM((RPW,), jnp.int32),
            pltpu.SemaphoreType.DMA,
        ],
    )
    def sc_kernel(x_hbm, coeff_hbm, idx_hbm, rhs_hbm, out_hbm,
                  idx_v, coeff_v, rhs_v, gath_v, acc_v, out_v, sem):
        wid = lax.axis_index("s") * NC + lax.axis_index("c")
        base = wid * RPW
        pltpu.sync_copy(idx_hbm, idx_v)
        pltpu.sync_copy(coeff_hbm, coeff_v)
        pltpu.sync_copy(rhs_hbm, rhs_v)
        rhs_vec = rhs_v[...]
        lane = lax.iota(jnp.int32, _LANES)

        def group(g, carry):
            row0 = base + g * G
            for r in range(G):
                for j in range(NCH):
                    pltpu.make_async_copy(
                        x_hbm.at[row0 + r].at[idx_v.at[j]],
                        gath_v.at[r * NCH + j],
                        sem,
                    ).start()
            for r in range(G):
                for j in range(NCH):
                    pltpu.make_async_copy(
                        x_hbm.at[row0 + r].at[idx_v.at[j]],
                        gath_v.at[r * NCH + j],
                        sem,
                    ).wait()
            for r in range(G):
                acc = jnp.zeros((_LANES,), jnp.float32)
                for j in range(NCH):
                    for u in range(VPC):
                        c = (j * VPC + u) * _LANES
                        acc = acc + (gath_v[r * NCH + j, pl.ds(u * _LANES, _LANES)]
                                     * coeff_v[pl.ds(c, _LANES)])
                acc_v[pl.ds(r * _LANES, _LANES)] = acc
            row_base = lane * _LANES
            out_vec = jnp.zeros((_LANES,), jnp.float32)
            for j in range(_LANES):
                out_vec = out_vec + plsc.load_gather(acc_v, [row_base + j])
            ok = (out_vec <= rhs_vec).astype(jnp.int32)
            out_v[pl.ds(g * G, G)] = ok
            return carry

        lax.fori_loop(0, NG, group, None)
        pltpu.sync_copy(out_v, out_hbm.at[pl.ds(base, RPW)])

    return sc_kernel


def kernel(x, coeff_tensor, indices_tensor, rhs):
    B, V = x.shape
    T = indices_tensor.shape[0]
    idx2 = indices_tensor.reshape(T // _CHUNK, _CHUNK)
    rhs_arr = jnp.full((_LANES,), rhs, dtype=jnp.float32)
    out = _build(B, V, T)(x, coeff_tensor, idx2, rhs_arr)
    return out.astype(bool)


# speedup vs baseline: 1.7746x; 1.7746x over previous
import functools

import jax
import jax.numpy as jnp
from jax import lax
from jax.experimental import pallas as pl
from jax.experimental.pallas import tpu as pltpu
from jax.experimental.pallas import tpu_sc as plsc

_LANES = 16
_KB = 1024


@functools.lru_cache(maxsize=None)
def _build_scatter(V, T):
    info = plsc.get_sparse_core_info()
    NC, NS = info.num_cores, info.num_subcores
    NW = NC * NS
    TPW = T // NW
    Vp = ((V + NS * 128 - 1) // (NS * 128)) * NS * 128
    CS = Vp // NS
    assert CS % 128 == 0

    mesh = plsc.VectorSubcoreMesh(core_axis_name="c", subcore_axis_name="s")

    @functools.partial(
        pl.kernel,
        out_type=jax.ShapeDtypeStruct((NC, Vp), jnp.float32),
        mesh=mesh,
        compiler_params=pltpu.CompilerParams(needs_layout_passes=False),
        scratch_types=[
            pltpu.VMEM((TPW,), jnp.int32),
            pltpu.VMEM((TPW,), jnp.float32),
            pltpu.VMEM((CS,), jnp.float32),
            pltpu.VMEM_SHARED((Vp,), jnp.float32),
        ],
    )
    def scatter_kernel(idx_hbm, coeff_hbm, out_hbm, idx_w, coeff_w, zer_v, s_sh):
        cid = lax.axis_index("c")
        sid = lax.axis_index("s")
        row = sid * NC + cid
        pltpu.sync_copy(idx_hbm.at[row], idx_w)
        pltpu.sync_copy(coeff_hbm.at[row], coeff_w)

        z = jnp.zeros((_LANES,), jnp.float32)

        def zero(i, carry):
            zer_v[pl.ds(i * _LANES, _LANES)] = z
            return carry

        lax.fori_loop(0, CS // _LANES, zero, None)
        pltpu.sync_copy(zer_v, s_sh.at[pl.ds(sid * CS, CS)])
        plsc.subcore_barrier()
        pltpu.sync_copy(coeff_w, s_sh.at[idx_w], add=True)
        plsc.subcore_barrier()
        pltpu.sync_copy(s_sh.at[pl.ds(sid * CS, CS)],
                        out_hbm.at[cid].at[pl.ds(sid * CS, CS)])

    return scatter_kernel


@functools.lru_cache(maxsize=None)
def _build_matvec(B, V, NC, Vp):
    grid = (Vp + _KB - 1) // _KB

    def body(x_ref, s_ref, rhs_ref, out_ref, acc_ref):
        k = pl.program_id(0)

        @pl.when(k == 0)
        def _():
            acc_ref[...] = jnp.zeros_like(acc_ref)

        limit = V - k * _KB
        lane = jax.lax.broadcasted_iota(jnp.int32, (1, _KB), 1)
        sb = s_ref[0:1, :] + s_ref[1:2, :]
        sb = jnp.where(lane < limit, sb, 0.0)
        xb = jnp.where(lane < limit, x_ref[...], 0.0)
        acc_ref[...] += xb * sb

        @pl.when(k == grid - 1)
        def _():
            lhs = jnp.sum(acc_ref[...], axis=1)
            out_ref[...] = (lhs <= rhs_ref[0]).astype(jnp.int32)

    return pl.pallas_call(
        body,
        grid=(grid,),
        out_shape=jax.ShapeDtypeStruct((B,), jnp.int32),
        in_specs=[
            pl.BlockSpec((B, _KB), lambda k: (0, k)),
            pl.BlockSpec((NC, _KB), lambda k: (0, k)),
            pl.BlockSpec(memory_space=pltpu.SMEM),
        ],
        out_specs=pl.BlockSpec((B,), lambda k: (0,)),
        scratch_shapes=[pltpu.VMEM((B, _KB), jnp.float32)],
        compiler_params=pltpu.CompilerParams(
            dimension_semantics=("arbitrary",)),
    )


def kernel(x, coeff_tensor, indices_tensor, rhs):
    B, V = x.shape
    T = indices_tensor.shape[0]
    info = plsc.get_sparse_core_info()
    NW = info.num_cores * info.num_subcores
    idx2 = indices_tensor.reshape(NW, T // NW)
    coeff2 = coeff_tensor.reshape(NW, T // NW)
    s = _build_scatter(V, T)(idx2, coeff2)
    rhs_arr = jnp.full((1,), rhs, dtype=jnp.float32)
    out = _build_matvec(B, V, s.shape[0], s.shape[1])(x, s, rhs_arr)
    return out.astype(bool)


# speedup vs baseline: 1.8238x; 1.0277x over previous
import functools

import jax
import jax.numpy as jnp
from jax import lax
from jax.experimental import pallas as pl
from jax.experimental.pallas import tpu as pltpu
from jax.experimental.pallas import tpu_sc as plsc

_LANES = 16
_KB = 2048
_NN = 8


@functools.lru_cache(maxsize=None)
def _build_scatter(V, T):
    info = plsc.get_sparse_core_info()
    NC, NS = info.num_cores, info.num_subcores
    NW = NC * NS
    TPW = T // NW
    Vp = ((V + NS * 128 - 1) // (NS * 128)) * NS * 128
    CS = Vp // NS
    assert CS % 128 == 0

    mesh = plsc.VectorSubcoreMesh(core_axis_name="c", subcore_axis_name="s")

    @functools.partial(
        pl.kernel,
        out_type=jax.ShapeDtypeStruct((NC, Vp), jnp.float32),
        mesh=mesh,
        compiler_params=pltpu.CompilerParams(needs_layout_passes=False),
        scratch_types=[
            pltpu.VMEM((TPW,), jnp.int32),
            pltpu.VMEM((TPW,), jnp.float32),
            pltpu.VMEM((CS,), jnp.float32),
            pltpu.VMEM_SHARED((Vp,), jnp.float32),
        ],
    )
    def scatter_kernel(idx_hbm, coeff_hbm, out_hbm, idx_w, coeff_w, zer_v, s_sh):
        cid = lax.axis_index("c")
        sid = lax.axis_index("s")
        row = sid * NC + cid
        pltpu.sync_copy(idx_hbm.at[row], idx_w)
        pltpu.sync_copy(coeff_hbm.at[row], coeff_w)

        z = jnp.zeros((_LANES,), jnp.float32)

        def zero(i, carry):
            zer_v[pl.ds(i * _LANES, _LANES)] = z
            return carry

        lax.fori_loop(0, CS // _LANES, zero, None)
        pltpu.sync_copy(zer_v, s_sh.at[pl.ds(sid * CS, CS)])
        plsc.subcore_barrier()
        pltpu.sync_copy(coeff_w, s_sh.at[idx_w], add=True)
        plsc.subcore_barrier()
        pltpu.sync_copy(s_sh.at[pl.ds(sid * CS, CS)],
                        out_hbm.at[cid].at[pl.ds(sid * CS, CS)])

    return scatter_kernel


@functools.lru_cache(maxsize=None)
def _build_matvec(B, V, NC, Vp):
    grid = Vp // _KB
    rem = V - (grid - 1) * _KB
    dims = (((1,), (1,)), ((), ()))

    def body(x_ref, s_ref, rhs_ref, out_ref, acc_ref):
        k = pl.program_id(0)

        @pl.when(k == 0)
        def _():
            acc_ref[...] = jnp.zeros_like(acc_ref)

        sb = s_ref[0:1, :] + s_ref[1:2, :]
        sbm = jnp.broadcast_to(sb, (_NN, _KB))

        @pl.when(k < grid - 1)
        def _():
            acc_ref[...] += lax.dot_general(
                x_ref[...], sbm, dims, preferred_element_type=jnp.float32)

        @pl.when(k == grid - 1)
        def _():
            lane = lax.broadcasted_iota(jnp.int32, (B, _KB), 1)
            xb = jnp.where(lane < rem, x_ref[...], 0.0)
            acc = acc_ref[...] + lax.dot_general(
                xb, sbm, dims, preferred_element_type=jnp.float32)
            lhs = acc[:, 0]
            out_ref[...] = (lhs <= rhs_ref[0]).astype(jnp.int32)

    return pl.pallas_call(
        body,
        grid=(grid,),
        out_shape=jax.ShapeDtypeStruct((B,), jnp.int32),
        in_specs=[
            pl.BlockSpec((B, _KB), lambda k: (0, k)),
            pl.BlockSpec((NC, _KB), lambda k: (0, k)),
            pl.BlockSpec(memory_space=pltpu.SMEM),
        ],
        out_specs=pl.BlockSpec((B,), lambda k: (0,)),
        scratch_shapes=[pltpu.VMEM((B, _NN), jnp.float32)],
        compiler_params=pltpu.CompilerParams(
            dimension_semantics=("arbitrary",)),
    )


def kernel(x, coeff_tensor, indices_tensor, rhs):
    B, V = x.shape
    T = indices_tensor.shape[0]
    info = plsc.get_sparse_core_info()
    NW = info.num_cores * info.num_subcores
    idx2 = indices_tensor.reshape(NW, T // NW)
    coeff2 = coeff_tensor.reshape(NW, T // NW)
    s = _build_scatter(V, T)(idx2, coeff2)
    rhs_arr = jnp.full((1,), rhs, dtype=jnp.float32)
    out = _build_matvec(B, V, s.shape[0], s.shape[1])(x, s, rhs_arr)
    return out.astype(bool)
